# baseline (speedup 1.0000x reference)
import functools

import jax
import jax.numpy as jnp
from jax import lax
from jax.experimental import pallas as pl
from jax.experimental.pallas import tpu as pltpu
from jax.experimental.pallas import tpu_sc as plsc

SIGNAL = 8192
DIM = 768
BATCH = 4
SPLIT = 2

NC = 2
NS = 16
L = 16
NW = NC * NS
ROWS_PER_W = SIGNAL // NW
CHUNK = 32
NCHUNK = ROWS_PER_W // CHUNK
NSEG = DIM // L
NB_SC = BATCH - SPLIT
NSTEP = NCHUNK * NB_SC
NBUF = 3

_mesh = plsc.VectorSubcoreMesh(core_axis_name="c", subcore_axis_name="s")


def _add_into(dst, src):
    @plsc.parallel_loop(0, CHUNK * NSEG, unroll=8)
    def _(i):
        r = i // NSEG
        j = i - r * NSEG
        sl = pl.ds(j * L, L)
        dst[r, sl] = dst[r, sl] + src[r, sl]


@functools.partial(
    pl.kernel,
    mesh=_mesh,
    out_type=jax.ShapeDtypeStruct((NB_SC, SIGNAL, DIM), jnp.float32),
    scratch_types=[
        pltpu.VMEM((CHUNK, DIM), jnp.float32),
        pltpu.VMEM((CHUNK, DIM), jnp.float32),
        pltpu.VMEM((CHUNK, DIM), jnp.float32),
        pltpu.VMEM((CHUNK, DIM), jnp.float32),
        pltpu.VMEM((CHUNK, DIM), jnp.float32),
        pltpu.SemaphoreType.DMA,
        pltpu.SemaphoreType.DMA,
        pltpu.SemaphoreType.DMA,
        pltpu.SemaphoreType.DMA,
        pltpu.SemaphoreType.DMA,
        pltpu.SemaphoreType.DMA,
        pltpu.SemaphoreType.DMA,
        pltpu.SemaphoreType.DMA,
    ],
)
def _pos_add_sc(patches_hbm, pos_hbm, out_hbm,
                pos0, pos1, buf0, buf1, buf2,
                psem0, psem1, lsem0, lsem1, lsem2, ssem0, ssem1, ssem2):
    wid = lax.axis_index("s") * NC + lax.axis_index("c")
    base_r = wid * ROWS_PER_W

    pos_v = (pos0, pos1)
    buf = (buf0, buf1, buf2)
    psem = (psem0, psem1)
    lsem = (lsem0, lsem1, lsem2)
    ssem = (ssem0, ssem1, ssem2)

    def rows(c):
        return pl.ds(base_r + c * CHUNK, CHUNK)

    pos_d = [None] * NCHUNK
    load_d = [None] * NSTEP
    store_d = [None] * NSTEP

    pos_d[0] = pltpu.async_copy(pos_hbm.at[rows(0)], pos_v[0], psem[0])
    for p in range(NBUF - 1):
        cp, bp = divmod(p, NB_SC)
        load_d[p] = pltpu.async_copy(patches_hbm.at[SPLIT + bp, rows(cp)],
                                     buf[p % NBUF], lsem[p % NBUF])

    for t in range(NSTEP):
        c, b = divmod(t, NB_SC)
        s = t % NBUF
        nt = t + NBUF - 1
        if nt < NSTEP:
            ns = nt % NBUF
            if t - 1 >= 0:
                store_d[t - 1].wait()
            c2, b2 = divmod(nt, NB_SC)
            load_d[nt] = pltpu.async_copy(
                patches_hbm.at[SPLIT + b2, rows(c2)], buf[ns], lsem[ns])
        if b == 0:
            pos_d[c].wait()
            if c + 1 < NCHUNK:
                pos_d[c + 1] = pltpu.async_copy(
                    pos_hbm.at[rows(c + 1)],
                    pos_v[(c + 1) % 2], psem[(c + 1) % 2])
        load_d[t].wait()
        _add_into(buf[s], pos_v[c % 2])
        store_d[t] = pltpu.async_copy(
            buf[s], out_hbm.at[b, rows(c)], ssem[s])

    for t in range(NSTEP - NBUF, NSTEP):
        store_d[t].wait()


BS = 256


def _tc_body(patches_ref, pos_ref, out_ref):
    out_ref[0] = patches_ref[0] + pos_ref[...]


def _pos_add_tc(patches, pos_table):
    grid = (SIGNAL // BS, SPLIT)
    return pl.pallas_call(
        _tc_body,
        grid=grid,
        in_specs=[
            pl.BlockSpec((1, BS, DIM), lambda i, b: (b, i, 0)),
            pl.BlockSpec((BS, DIM), lambda i, b: (i, 0)),
        ],
        out_specs=pl.BlockSpec((1, BS, DIM), lambda i, b: (b, i, 0)),
        out_shape=jax.ShapeDtypeStruct((SPLIT, SIGNAL, DIM), jnp.float32),
    )(patches, pos_table)


def kernel(patches, pos_table):
    sc_out = _pos_add_sc(patches, pos_table)
    tc_out = _pos_add_tc(patches, pos_table)
    return jnp.concatenate([tc_out, sc_out], axis=0)


# speedup vs baseline: 1.6879x; 1.6879x over previous
import functools

import jax
import jax.numpy as jnp
from jax import lax
from jax.experimental import pallas as pl
from jax.experimental.pallas import tpu as pltpu
from jax.experimental.pallas import tpu_sc as plsc

SIGNAL = 8192
DIM = 768
BATCH = 4

NC = 2
NS = 16
L = 16
NW = NC * NS
ROWS_PER_W = SIGNAL // NW
CHUNK = 16
NCHUNK = ROWS_PER_W // CHUNK
NSEG = DIM // L
NVEC = CHUNK * NSEG

_mesh = plsc.VectorSubcoreMesh(core_axis_name="c", subcore_axis_name="s")


def _add_group(bufs, pos):
    @plsc.parallel_loop(0, NVEC, unroll=4)
    def _(i):
        r = i // NSEG
        j = i - r * NSEG
        sl = pl.ds(j * L, L)
        p = pos[r, sl]
        for b in range(BATCH):
            bufs[b][r, sl] = bufs[b][r, sl] + p


@functools.partial(
    pl.kernel,
    mesh=_mesh,
    out_type=jax.ShapeDtypeStruct((BATCH, SIGNAL, DIM), jnp.float32),
    scratch_types=(
        [pltpu.VMEM((CHUNK, DIM), jnp.float32)] * 2
        + [pltpu.VMEM((CHUNK, DIM), jnp.float32)] * 8
        + [pltpu.SemaphoreType.DMA] * 2
        + [pltpu.SemaphoreType.DMA] * 2
        + [pltpu.SemaphoreType.DMA] * 2
    ),
)
def _pos_add(patches_hbm, pos_hbm, out_hbm,
             pos0, pos1, a0, a1, a2, a3, b0, b1, b2, b3,
             psem0, psem1, lsem0, lsem1, ssem0, ssem1):
    wid = lax.axis_index("s") * NC + lax.axis_index("c")
    base_r = wid * ROWS_PER_W

    pos_v = (pos0, pos1)
    grp = ((a0, a1, a2, a3), (b0, b1, b2, b3))
    psem = (psem0, psem1)
    lsem = (lsem0, lsem1)
    ssem = (ssem0, ssem1)

    def rows(c):
        return pl.ds(base_r + c * CHUNK, CHUNK)

    pos_d = [None] * NCHUNK
    load_d = [[None] * BATCH for _ in range(NCHUNK)]
    store_d = [[None] * BATCH for _ in range(NCHUNK)]

    def issue_loads(c):
        gg = c % 2
        for b in range(BATCH):
            load_d[c][b] = pltpu.async_copy(
                patches_hbm.at[b, rows(c)], grp[gg][b], lsem[gg])

    pos_d[0] = pltpu.async_copy(pos_hbm.at[rows(0)], pos_v[0], psem[0])
    issue_loads(0)

    for c in range(NCHUNK):
        gg = c % 2
        if c + 1 < NCHUNK:
            if c - 1 >= 0:
                for b in range(BATCH):
                    store_d[c - 1][b].wait()
            issue_loads(c + 1)
        pos_d[c].wait()
        if c + 1 < NCHUNK:
            pos_d[c + 1] = pltpu.async_copy(
                pos_hbm.at[rows(c + 1)], pos_v[(c + 1) % 2], psem[(c + 1) % 2])
        for b in range(BATCH):
            load_d[c][b].wait()
        _add_group(grp[gg], pos_v[gg])
        for b in range(BATCH):
            store_d[c][b] = pltpu.async_copy(
                grp[gg][b], out_hbm.at[b, rows(c)], ssem[gg])

    for c in (NCHUNK - 2, NCHUNK - 1):
        for b in range(BATCH):
            store_d[c][b].wait()


def kernel(patches, pos_table):
    return _pos_add(patches, pos_table)
